# baseline (speedup 1.0000x reference)
import jax
import jax.numpy as jnp
from jax.experimental import pallas as pl
from jax.experimental.pallas import tpu as pltpu

_N = 4096
_BLK = 512
_GRID = _N // _BLK
_ETA = 1000000.0


def _body(Q_ref, AT_ref, x_ref, y_ref, c_ref, b_ref, il_ref, iu_ref,
          l_ref, u_ref, o_ref, acc_ref):
    i = pl.program_id(0)

    @pl.when(i == 0)
    def _init():
        acc_ref[0] = 0.0

    x = x_ref[...]
    y = y_ref[...]
    qx = jnp.dot(Q_ref[...], x, preferred_element_type=jnp.float32)
    aty = jnp.dot(AT_ref[...], y, preferred_element_type=jnp.float32)

    sl = pl.ds(i * _BLK, _BLK)
    xb = x_ref[sl, :]
    yb = y_ref[sl, :]
    cb = c_ref[sl, :]
    bb = b_ref[sl, :]

    pg = cb - aty + qx
    rc = jnp.maximum(pg, 0.0) * il_ref[sl, :] - jnp.maximum(-pg, 0.0) * iu_ref[sl, :]
    rcc = jnp.sum(jnp.where(rc > 0.0, l_ref[sl, :], u_ref[sl, :]) * rc)
    contrib = (jnp.sum(xb * qx) + jnp.sum(cb * xb)
               - jnp.sum(bb * yb) - rcc)
    acc_ref[0] = acc_ref[0] + contrib

    @pl.when(i == _GRID - 1)
    def _fin():
        o_ref[...] = jnp.full((1, 1), jnp.abs(acc_ref[0]) / _ETA,
                              dtype=jnp.float32)


def kernel(Q, A, AT, b, c, x, y, Iy, il, iu, l, u):
    del A, Iy
    c2 = c[:, None]
    b2 = b[:, None]
    vec = pl.BlockSpec((_N, 1), lambda i: (0, 0))
    out = pl.pallas_call(
        _body,
        grid=(_GRID,),
        in_specs=[
            pl.BlockSpec((_BLK, _N), lambda i: (i, 0)),
            pl.BlockSpec((_BLK, _N), lambda i: (i, 0)),
            vec, vec, vec, vec, vec, vec, vec, vec,
        ],
        out_specs=pl.BlockSpec((1, 1), lambda i: (0, 0)),
        out_shape=jax.ShapeDtypeStruct((1, 1), jnp.float32),
        scratch_shapes=[pltpu.SMEM((1,), jnp.float32)],
        compiler_params=pltpu.CompilerParams(
            dimension_semantics=("arbitrary",)),
    )(Q, AT, x, y, c2, b2, il, iu, l, u)
    return out


# speedup vs baseline: 1.0088x; 1.0088x over previous
import jax
import jax.numpy as jnp
from jax.experimental import pallas as pl
from jax.experimental.pallas import tpu as pltpu

_N = 4096
_BLK = 256
_GRID = _N // _BLK
_ETA = 1000000.0


def _body(Q_ref, AT_ref, x_ref, y_ref, c_ref, b_ref, il_ref, iu_ref,
          l_ref, u_ref, o_ref, acc_ref):
    i = pl.program_id(0)

    @pl.when(i == 0)
    def _init():
        acc_ref[0] = 0.0

    x = x_ref[...]
    y = y_ref[...]
    qx = jnp.dot(Q_ref[...], x, preferred_element_type=jnp.float32)
    aty = jnp.dot(AT_ref[...], y, preferred_element_type=jnp.float32)

    sl = pl.ds(i * _BLK, _BLK)
    xb = x_ref[sl, :]
    yb = y_ref[sl, :]
    cb = c_ref[sl, :]
    bb = b_ref[sl, :]

    pg = cb - aty + qx
    rc = jnp.maximum(pg, 0.0) * il_ref[sl, :] - jnp.maximum(-pg, 0.0) * iu_ref[sl, :]
    rcc = jnp.sum(jnp.where(rc > 0.0, l_ref[sl, :], u_ref[sl, :]) * rc)
    contrib = (jnp.sum(xb * qx) + jnp.sum(cb * xb)
               - jnp.sum(bb * yb) - rcc)
    acc_ref[0] = acc_ref[0] + contrib

    @pl.when(i == _GRID - 1)
    def _fin():
        o_ref[...] = jnp.full((1, 1), jnp.abs(acc_ref[0]) / _ETA,
                              dtype=jnp.float32)


def kernel(Q, A, AT, b, c, x, y, Iy, il, iu, l, u):
    del A, Iy
    c2 = c[:, None]
    b2 = b[:, None]
    vec = pl.BlockSpec((_N, 1), lambda i: (0, 0))
    out = pl.pallas_call(
        _body,
        grid=(_GRID,),
        in_specs=[
            pl.BlockSpec((_BLK, _N), lambda i: (i, 0)),
            pl.BlockSpec((_BLK, _N), lambda i: (i, 0)),
            vec, vec, vec, vec, vec, vec, vec, vec,
        ],
        out_specs=pl.BlockSpec((1, 1), lambda i: (0, 0)),
        out_shape=jax.ShapeDtypeStruct((1, 1), jnp.float32),
        scratch_shapes=[pltpu.SMEM((1,), jnp.float32)],
        compiler_params=pltpu.CompilerParams(
            dimension_semantics=("arbitrary",)),
    )(Q, AT, x, y, c2, b2, il, iu, l, u)
    return out
